# baseline (speedup 1.0000x reference)
import functools

import jax
import jax.numpy as jnp
from jax import lax
from jax.experimental import pallas as pl
from jax.experimental.pallas import tpu as pltpu
from jax.experimental.pallas import tpu_sc as plsc

BATCH = 4096
MAXLEN = 200
EMBED = 64
LANES = 16
NW = 32
ROWS_W = BATCH // NW
NB = 4
HALF = MAXLEN // 2

_mesh = plsc.VectorSubcoreMesh(core_axis_name="c", subcore_axis_name="s")


@functools.partial(
    pl.kernel,
    mesh=_mesh,
    compiler_params=pltpu.CompilerParams(use_tc_tiling_on_sc=False),
    out_type=jax.ShapeDtypeStruct((BATCH * MAXLEN, EMBED), jnp.float32),
    scratch_types=(
        [pltpu.VMEM((ROWS_W, 2, HALF), jnp.int32)]
        + [pltpu.VMEM((MAXLEN, EMBED), jnp.float32)]
        + [pltpu.VMEM((MAXLEN, EMBED), jnp.float32) for _ in range(NB)]
        + [pltpu.SemaphoreType.DMA for _ in range(NB)]
        + [pltpu.SemaphoreType.DMA for _ in range(NB)]
    ),
)
def _tok_pos_embed(x_hbm, tok_hbm, pos_hbm, out_hbm,
                   idx_v, pos_v, r0, r1, r2, r3,
                   g0, g1, g2, g3, s0, s1, s2, s3):
    rows = (r0, r1, r2, r3)
    gsem = (g0, g1, g2, g3)
    ssem = (s0, s1, s2, s3)

    wid = lax.axis_index("s") * 2 + lax.axis_index("c")
    base = wid * ROWS_W

    pltpu.sync_copy(x_hbm.at[pl.ds(base, ROWS_W)], idx_v)
    pltpu.sync_copy(pos_hbm, pos_v)

    def issue_gather(b, c):
        pltpu.async_copy(tok_hbm.at[idx_v.at[c, 0]],
                         rows[b].at[pl.ds(0, HALF)], gsem[b])
        pltpu.async_copy(tok_hbm.at[idx_v.at[c, 1]],
                         rows[b].at[pl.ds(HALF, HALF)], gsem[b])

    def wait_gather(b):
        pltpu.make_async_copy(tok_hbm.at[pl.ds(0, MAXLEN)], rows[b],
                              gsem[b]).wait()

    def wait_scatter(b):
        pltpu.make_async_copy(rows[b], out_hbm.at[pl.ds(0, MAXLEN)],
                              ssem[b]).wait()

    def add_pos(b):
        rbuf = rows[b]

        def rbody(r, carry):
            for k in range(EMBED // LANES):
                sl = pl.ds(k * LANES, LANES)
                plsc.addupdate(rbuf.at[r, sl], pos_v[r, sl])
            return carry

        lax.fori_loop(0, MAXLEN, rbody, 0, unroll=2)

    issue_gather(0, 0)
    issue_gather(1, 1)

    def outer(g, carry):
        for b in range(NB):
            c = g * NB + b
            nxt = c + 2
            nb_ = (b + 2) % NB

            @pl.when(nxt < ROWS_W)
            def _():
                @pl.when(c >= 2)
                def _():
                    wait_scatter(nb_)
                issue_gather(nb_, nxt)

            wait_gather(b)
            add_pos(b)
            pltpu.async_copy(rows[b],
                             out_hbm.at[pl.ds((base + c) * MAXLEN, MAXLEN)],
                             ssem[b])
        return carry

    lax.fori_loop(0, ROWS_W // NB, outer, 0)

    for b in range(NB):
        wait_scatter(b)


def kernel(x, token_table, pos_table):
    x3 = jnp.reshape(x.astype(jnp.int32), (BATCH, 2, HALF))
    out = _tok_pos_embed(x3, token_table, pos_table)
    return jnp.reshape(out, (BATCH, MAXLEN, EMBED))
